# baseline (speedup 1.0000x reference)
import functools

import jax
import jax.numpy as jnp
from jax import lax
from jax.experimental import pallas as pl
from jax.experimental.pallas import tpu as pltpu
from jax.experimental.pallas import tpu_sc as plsc

RADIUS = 1.0
EPS = 1e-4
NY = 201
NUM_DIMS = 4
N_ROWS = 200000

M_GRID = 2048
H = 2.0 / M_GRID
TAB_ROWS = M_GRID + 1
TAB_ROWS_PAD = 2056
TAB_FLAT = TAB_ROWS * 12
TAB_FLAT_PAD = 24592

NW = 32
CH = 6272
NCH_FULL = CH // 16
LAST_BASE = (NW - 1) * CH
CH_LAST = N_ROWS - LAST_BASE
NCH_LAST = CH_LAST // 16
N_GROUPS = 1563
OUT_WORDS = N_GROUPS * 512
CH_OUT = 49 * 512
CH_OUT_LAST = OUT_WORDS - (NW - 1) * CH_OUT


def _table_body(icpt_ref, d_ref, m1_ref, m2_ref):
    b = icpt_ref[...]
    bm = b - jnp.mean(b, axis=0, keepdims=True)
    jj = lax.broadcasted_iota(jnp.int32, (NY, NUM_DIMS), 0).astype(jnp.float32)
    y = jj * 0.01 - 1.0
    e = jnp.exp(-50.0 * bm)
    c0 = e
    c1 = y * e
    c2 = (-0.5 * y * y - bm) * e
    mm = lax.broadcasted_iota(jnp.int32, (TAB_ROWS_PAD, NY), 0).astype(jnp.float32)
    jb = lax.broadcasted_iota(jnp.int32, (TAB_ROWS_PAD, NY), 1).astype(jnp.float32)
    xs = mm * H - 1.0
    yb = jb * 0.01 - 1.0
    diff = xs - yb
    a = jnp.exp(-25.0 * diff * diff)
    hi = jax.lax.Precision.HIGHEST
    d_ref[...] = jnp.dot(a, c0, precision=hi, preferred_element_type=jnp.float32)
    m1_ref[...] = jnp.dot(a, c1, precision=hi, preferred_element_type=jnp.float32)
    m2_ref[...] = jnp.dot(a, c2, precision=hi, preferred_element_type=jnp.float32)


_mesh = plsc.VectorSubcoreMesh(core_axis_name="c", subcore_axis_name="s")


@functools.partial(
    pl.kernel,
    mesh=_mesh,
    compiler_params=pltpu.CompilerParams(needs_layout_passes=False),
    out_type=(
        jax.ShapeDtypeStruct((OUT_WORDS,), jnp.float32),
        jax.ShapeDtypeStruct((N_ROWS,), jnp.float32),
    ),
    scratch_types=[
        pltpu.VMEM((CH * 4,), jnp.float32),
        pltpu.VMEM((TAB_FLAT_PAD,), jnp.float32),
        pltpu.VMEM((CH_OUT,), jnp.float32),
        pltpu.VMEM((CH,), jnp.float32),
    ],
)
def _sc_interp(x0, x1, x2, x3, tab_hbm, ch_hbm, f_hbm, x_v, tab_v, ch_v, f_v):
    wid = lax.axis_index("s") * 2 + lax.axis_index("c")
    base = wid * CH
    is_last = wid == NW - 1
    xs = (x0, x1, x2, x3)

    @pl.when(jnp.logical_not(is_last))
    def _():
        for d in range(NUM_DIMS):
            pltpu.sync_copy(xs[d].at[pl.ds(base, CH)], x_v.at[pl.ds(d * CH, CH)])

    @pl.when(is_last)
    def _():
        for d in range(NUM_DIMS):
            pltpu.sync_copy(
                xs[d].at[pl.ds(LAST_BASE, CH_LAST)], x_v.at[pl.ds(d * CH, CH_LAST)]
            )

    pltpu.sync_copy(tab_hbm, tab_v)
    nch = jnp.where(is_last, NCH_LAST, NCH_FULL)

    @plsc.parallel_loop(0, nch, unroll=2)
    def body(i):
        out0 = (i // 8) * 512 + (i % 8) * 16
        facc = jnp.zeros((16,), jnp.float32)
        for d in range(NUM_DIMS):
            x = x_v[pl.ds(d * CH + i * 16, 16)]
            xc = jnp.clip(x, -RADIUS + EPS, RADIUS - EPS)
            p = (xc + 1.0) * (M_GRID / 2.0)
            ji = p.astype(jnp.int32)
            t = p - ji.astype(jnp.float32)
            idx0 = ji * 12 + (d * 3)
            acc = [None, None, None]
            for k in range(3):
                v0 = plsc.load_gather(tab_v, [idx0 + k])
                v1 = plsc.load_gather(tab_v, [idx0 + (12 + k)])
                acc[k] = v0 + t * (v1 - v0)
            rd = 1.0 / acc[0]
            ch = acc[1] * rd
            fd = xc * ch + acc[2] * rd - 0.5 * xc * xc
            facc = facc + fd
            ch_v[pl.ds(out0 + d * 128, 16)] = ch
        f_v[pl.ds(i * 16, 16)] = facc

    @pl.when(jnp.logical_not(is_last))
    def _():
        pltpu.sync_copy(ch_v, ch_hbm.at[pl.ds(wid * CH_OUT, CH_OUT)])
        pltpu.sync_copy(f_v, f_hbm.at[pl.ds(base, CH)])

    @pl.when(is_last)
    def _():
        pltpu.sync_copy(
            ch_v.at[pl.ds(0, CH_OUT_LAST)],
            ch_hbm.at[pl.ds((NW - 1) * CH_OUT, CH_OUT_LAST)],
        )
        pltpu.sync_copy(f_v.at[pl.ds(0, CH_LAST)], f_hbm.at[pl.ds(LAST_BASE, CH_LAST)])


def kernel(X, intercepts):
    dt, m1t, m2t = pl.pallas_call(
        _table_body,
        out_shape=[jax.ShapeDtypeStruct((TAB_ROWS_PAD, NUM_DIMS), jnp.float32)] * 3,
    )(intercepts)
    t3 = jnp.stack([dt, m1t, m2t], axis=-1)
    tab = jnp.concatenate(
        [t3[:TAB_ROWS].reshape(-1), jnp.zeros((TAB_FLAT_PAD - TAB_FLAT,), jnp.float32)]
    )
    ch_flat, f_f = _sc_interp(X[:, 0], X[:, 1], X[:, 2], X[:, 3], tab)
    choice = (
        ch_flat.reshape(N_GROUPS, NUM_DIMS, 128)
        .transpose(0, 2, 1)
        .reshape(N_GROUPS * 128, NUM_DIMS)[:N_ROWS]
    )
    return choice, f_f


# speedup vs baseline: 1.0327x; 1.0327x over previous
import functools

import jax
import jax.numpy as jnp
from jax import lax
from jax.experimental import pallas as pl
from jax.experimental.pallas import tpu as pltpu
from jax.experimental.pallas import tpu_sc as plsc

RADIUS = 1.0
EPS = 1e-4
NY = 201
NUM_DIMS = 4
N_ROWS = 200000

M_GRID = 1024
H = 2.0 / M_GRID
TAB_ROWS = M_GRID + 1
TAB_ROWS_PAD = 1032
TAB_FLAT = TAB_ROWS * 12
TAB_FLAT_PAD = 12304

NW = 32
CH = 6272
NCH_FULL = CH // 16
LAST_BASE = (NW - 1) * CH
CH_LAST = N_ROWS - LAST_BASE
NCH_LAST = 352
N_GROUPS = 1563
OUT_WORDS = N_GROUPS * 512
CH_OUT = 49 * 512
CH_OUT_LAST = OUT_WORDS - (NW - 1) * CH_OUT


def _table_body(icpt_ref, tab_ref):
    b = icpt_ref[...]
    bm = b - jnp.mean(b, axis=0, keepdims=True)
    jj = lax.broadcasted_iota(jnp.int32, (NY, NUM_DIMS), 0).astype(jnp.float32)
    y = jj * 0.01 - 1.0
    e = jnp.exp(-50.0 * bm)
    c0 = e
    c1 = y * e
    c2 = (-0.5 * y * y - bm) * e
    cols = []
    for d in range(NUM_DIMS):
        cols += [c0[:, d : d + 1], c1[:, d : d + 1], c2[:, d : d + 1]]
    c = jnp.concatenate(cols, axis=1)
    mm = lax.broadcasted_iota(jnp.int32, (TAB_ROWS_PAD, NY), 0).astype(jnp.float32)
    jb = lax.broadcasted_iota(jnp.int32, (TAB_ROWS_PAD, NY), 1).astype(jnp.float32)
    xs = mm * H - 1.0
    yb = jb * 0.01 - 1.0
    diff = xs - yb
    a = jnp.exp(-25.0 * diff * diff)
    hi = jax.lax.Precision.HIGHEST
    tab_ref[...] = jnp.dot(a, c, precision=hi, preferred_element_type=jnp.float32)


_mesh = plsc.VectorSubcoreMesh(core_axis_name="c", subcore_axis_name="s")


@functools.partial(
    pl.kernel,
    mesh=_mesh,
    compiler_params=pltpu.CompilerParams(needs_layout_passes=False),
    out_type=(
        jax.ShapeDtypeStruct((OUT_WORDS,), jnp.float32),
        jax.ShapeDtypeStruct((N_ROWS,), jnp.float32),
    ),
    scratch_types=[
        pltpu.VMEM((CH_OUT,), jnp.float32),
        pltpu.VMEM((TAB_FLAT_PAD,), jnp.float32),
        pltpu.VMEM((CH_OUT,), jnp.float32),
        pltpu.VMEM((CH,), jnp.float32),
    ],
)
def _sc_interp(x_hbm, tab_hbm, ch_hbm, f_hbm, x_v, tab_v, ch_v, f_v):
    wid = lax.axis_index("s") * 2 + lax.axis_index("c")
    base = wid * CH
    is_last = wid == NW - 1

    @pl.when(jnp.logical_not(is_last))
    def _():
        pltpu.sync_copy(x_hbm.at[pl.ds(wid * CH_OUT, CH_OUT)], x_v)

    @pl.when(is_last)
    def _():
        pltpu.sync_copy(
            x_hbm.at[pl.ds((NW - 1) * CH_OUT, CH_OUT_LAST)],
            x_v.at[pl.ds(0, CH_OUT_LAST)],
        )

    pltpu.sync_copy(tab_hbm, tab_v)
    nch = jnp.where(is_last, NCH_LAST, NCH_FULL)

    @plsc.parallel_loop(0, nch, unroll=2)
    def body(i):
        out0 = (i // 8) * 512 + (i % 8) * 16
        facc = jnp.zeros((16,), jnp.float32)
        for d in range(NUM_DIMS):
            x = x_v[pl.ds(out0 + d * 128, 16)]
            xc = jnp.clip(x, -RADIUS + EPS, RADIUS - EPS)
            p = (xc + 1.0) * (M_GRID / 2.0)
            ji = p.astype(jnp.int32)
            t = p - ji.astype(jnp.float32)
            idx0 = ji * 12 + (d * 3)
            acc = [None, None, None]
            for k in range(3):
                v0 = plsc.load_gather(tab_v, [idx0 + k])
                v1 = plsc.load_gather(tab_v, [idx0 + (12 + k)])
                acc[k] = v0 + t * (v1 - v0)
            rd = 1.0 / acc[0]
            ch = acc[1] * rd
            fd = xc * ch + acc[2] * rd - 0.5 * xc * xc
            facc = facc + fd
            ch_v[pl.ds(out0 + d * 128, 16)] = ch
        f_v[pl.ds(i * 16, 16)] = facc

    @pl.when(jnp.logical_not(is_last))
    def _():
        pltpu.sync_copy(ch_v, ch_hbm.at[pl.ds(wid * CH_OUT, CH_OUT)])
        pltpu.sync_copy(f_v, f_hbm.at[pl.ds(base, CH)])

    @pl.when(is_last)
    def _():
        pltpu.sync_copy(
            ch_v.at[pl.ds(0, CH_OUT_LAST)],
            ch_hbm.at[pl.ds((NW - 1) * CH_OUT, CH_OUT_LAST)],
        )
        pltpu.sync_copy(f_v.at[pl.ds(0, CH_LAST)], f_hbm.at[pl.ds(LAST_BASE, CH_LAST)])


def kernel(X, intercepts):
    tabp = pl.pallas_call(
        _table_body,
        out_shape=jax.ShapeDtypeStruct((TAB_ROWS_PAD, 12), jnp.float32),
    )(intercepts)
    tab = jnp.concatenate(
        [
            tabp[:TAB_ROWS].reshape(-1),
            jnp.zeros((TAB_FLAT_PAD - TAB_FLAT,), jnp.float32),
        ]
    )
    xp = jnp.pad(X, ((0, N_GROUPS * 128 - N_ROWS), (0, 0)))
    x_flat = xp.reshape(N_GROUPS, 128, NUM_DIMS).transpose(0, 2, 1).reshape(-1)
    ch_flat, f_f = _sc_interp(x_flat, tab)
    choice = (
        ch_flat.reshape(N_GROUPS, NUM_DIMS, 128)
        .transpose(0, 2, 1)
        .reshape(N_GROUPS * 128, NUM_DIMS)[:N_ROWS]
    )
    return choice, f_f


# speedup vs baseline: 1.1983x; 1.1604x over previous
import functools

import jax
import jax.numpy as jnp
from jax import lax
from jax.experimental import pallas as pl
from jax.experimental.pallas import tpu as pltpu
from jax.experimental.pallas import tpu_sc as plsc

RADIUS = 1.0
EPS = 1e-4
NY = 201
NUM_DIMS = 4
N_ROWS = 200000

M_GRID = 1024
H = 2.0 / M_GRID
TAB_ROWS = M_GRID + 1
TAB_ROWS_PAD = 1032
TAB_FLAT = TAB_ROWS * 12
TAB_FLAT_PAD = 12304

NW = 32
CH = 6272
NCH_FULL = CH // 16
LAST_BASE = (NW - 1) * CH
CH_LAST = N_ROWS - LAST_BASE
NCH_LAST = CH_LAST // 16
N_GROUPS = 1563
OUT_WORDS = N_GROUPS * 512
CH_OUT = 49 * 512
CH_OUT_LAST = OUT_WORDS - (NW - 1) * CH_OUT


def _table_body(icpt_ref, d_ref, m1_ref, m2_ref):
    b = icpt_ref[...]
    bm = b - jnp.mean(b, axis=0, keepdims=True)
    jj = lax.broadcasted_iota(jnp.int32, (NY, NUM_DIMS), 0).astype(jnp.float32)
    y = jj * 0.01 - 1.0
    e = jnp.exp(-50.0 * bm)
    c0 = e
    c1 = y * e
    c2 = (-0.5 * y * y - bm) * e
    mm = lax.broadcasted_iota(jnp.int32, (TAB_ROWS_PAD, NY), 0).astype(jnp.float32)
    jb = lax.broadcasted_iota(jnp.int32, (TAB_ROWS_PAD, NY), 1).astype(jnp.float32)
    xs = mm * H - 1.0
    yb = jb * 0.01 - 1.0
    diff = xs - yb
    a = jnp.exp(-25.0 * diff * diff)
    hi = jax.lax.Precision.HIGHEST
    d_ref[...] = jnp.dot(a, c0, precision=hi, preferred_element_type=jnp.float32)
    m1_ref[...] = jnp.dot(a, c1, precision=hi, preferred_element_type=jnp.float32)
    m2_ref[...] = jnp.dot(a, c2, precision=hi, preferred_element_type=jnp.float32)


_mesh = plsc.VectorSubcoreMesh(core_axis_name="c", subcore_axis_name="s")


@functools.partial(
    pl.kernel,
    mesh=_mesh,
    compiler_params=pltpu.CompilerParams(needs_layout_passes=False),
    out_type=(
        jax.ShapeDtypeStruct((OUT_WORDS,), jnp.float32),
        jax.ShapeDtypeStruct((N_ROWS,), jnp.float32),
    ),
    scratch_types=[
        pltpu.VMEM((CH * 4,), jnp.float32),
        pltpu.VMEM((TAB_FLAT_PAD,), jnp.float32),
        pltpu.VMEM((CH_OUT,), jnp.float32),
        pltpu.VMEM((CH,), jnp.float32),
    ],
)
def _sc_interp(x0, x1, x2, x3, tab_hbm, ch_hbm, f_hbm, x_v, tab_v, ch_v, f_v):
    wid = lax.axis_index("s") * 2 + lax.axis_index("c")
    base = wid * CH
    is_last = wid == NW - 1
    xs_in = (x0, x1, x2, x3)

    @pl.when(jnp.logical_not(is_last))
    def _():
        for d in range(NUM_DIMS):
            pltpu.sync_copy(xs_in[d].at[pl.ds(base, CH)], x_v.at[pl.ds(d * CH, CH)])

    @pl.when(is_last)
    def _():
        for d in range(NUM_DIMS):
            pltpu.sync_copy(
                xs_in[d].at[pl.ds(LAST_BASE, CH_LAST)], x_v.at[pl.ds(d * CH, CH_LAST)]
            )

    pltpu.sync_copy(tab_hbm, tab_v)
    nch = jnp.where(is_last, NCH_LAST, NCH_FULL)

    @plsc.parallel_loop(0, nch, unroll=2)
    def body(i):
        out0 = (i // 8) * 512 + (i % 8) * 16
        facc = jnp.zeros((16,), jnp.float32)
        for d in range(NUM_DIMS):
            x = x_v[pl.ds(d * CH + i * 16, 16)]
            xc = jnp.clip(x, -RADIUS + EPS, RADIUS - EPS)
            p = (xc + 1.0) * (M_GRID / 2.0)
            ji = p.astype(jnp.int32)
            t = p - ji.astype(jnp.float32)
            idx0 = ji * 12 + (d * 3)
            acc = [None, None, None]
            for k in range(3):
                v0 = plsc.load_gather(tab_v, [idx0 + k])
                v1 = plsc.load_gather(tab_v, [idx0 + (12 + k)])
                acc[k] = v0 + t * (v1 - v0)
            rd = 1.0 / acc[0]
            ch = acc[1] * rd
            fd = xc * ch + acc[2] * rd - 0.5 * xc * xc
            facc = facc + fd
            ch_v[pl.ds(out0 + d * 128, 16)] = ch
        f_v[pl.ds(i * 16, 16)] = facc

    @pl.when(jnp.logical_not(is_last))
    def _():
        pltpu.sync_copy(ch_v, ch_hbm.at[pl.ds(wid * CH_OUT, CH_OUT)])
        pltpu.sync_copy(f_v, f_hbm.at[pl.ds(base, CH)])

    @pl.when(is_last)
    def _():
        pltpu.sync_copy(
            ch_v.at[pl.ds(0, CH_OUT_LAST)],
            ch_hbm.at[pl.ds((NW - 1) * CH_OUT, CH_OUT_LAST)],
        )
        pltpu.sync_copy(f_v.at[pl.ds(0, CH_LAST)], f_hbm.at[pl.ds(LAST_BASE, CH_LAST)])


def kernel(X, intercepts):
    dt, m1t, m2t = pl.pallas_call(
        _table_body,
        out_shape=[jax.ShapeDtypeStruct((TAB_ROWS_PAD, NUM_DIMS), jnp.float32)] * 3,
    )(intercepts)
    t3 = jnp.stack([dt, m1t, m2t], axis=-1)
    tab = jnp.concatenate(
        [t3[:TAB_ROWS].reshape(-1), jnp.zeros((TAB_FLAT_PAD - TAB_FLAT,), jnp.float32)]
    )
    ch_flat, f_f = _sc_interp(X[:, 0], X[:, 1], X[:, 2], X[:, 3], tab)
    choice = (
        ch_flat.reshape(N_GROUPS, NUM_DIMS, 128)
        .transpose(0, 2, 1)
        .reshape(N_GROUPS * 128, NUM_DIMS)[:N_ROWS]
    )
    return choice, f_f
